# baseline (speedup 1.0000x reference)
import jax
import jax.numpy as jnp
from jax.experimental import pallas as pl
from jax.experimental.pallas import tpu as pltpu

_B, _CIN, _SY, _SX = 8, 96, 32, 32
_IMGS = 2
_CPAD = 128
_HID = 1024
_OUT = 425
_PIX = _SY * _SX
_EPS = 1e-5


def _head_kernel(x_ref, w1_ref, g_ref, b_ref, m_ref, v_ref, w2_ref, b2_ref,
                 o_ref):
    cols = []
    for i in range(_IMGS):
        pieces = []
        for dy in range(3):
            for dx in range(3):
                patch = x_ref[i, dx, dy:dy + _SY, :, :].reshape(_PIX, _CIN)
                pieces.append(jnp.pad(patch, ((0, 0), (0, _CPAD - _CIN))))
        cols.append(jnp.concatenate(pieces, axis=1))
    col = jnp.concatenate(cols, axis=0)
    acc = jnp.dot(col, w1_ref[...], preferred_element_type=jnp.float32)
    scale = g_ref[...] * jax.lax.rsqrt(v_ref[...] + _EPS)
    shift = b_ref[...] - m_ref[...] * scale
    h = acc * scale + shift
    h = jnp.where(h >= 0, h, 0.1 * h)
    out = jnp.dot(h.astype(jnp.bfloat16), w2_ref[...],
                  preferred_element_type=jnp.float32) + b2_ref[...]
    o_ref[...] = out.reshape(_IMGS, _SY, _SX, _OUT)


def kernel(x, W1, gamma, beta, running_mean, running_var, W2, b2):
    xp = jnp.transpose(x, (0, 2, 3, 1))
    xp = jnp.pad(xp, ((0, 0), (1, 1), (1, 1), (0, 0))).astype(jnp.bfloat16)
    xs = jnp.stack([xp[:, :, dx:dx + _SX, :] for dx in range(3)], axis=1)

    w1 = jnp.transpose(W1, (2, 3, 1, 0)).reshape(9, _CIN, _HID)
    w1 = jnp.pad(w1, ((0, 0), (0, _CPAD - _CIN), (0, 0)))
    w1 = w1.reshape(9 * _CPAD, _HID).astype(jnp.bfloat16)
    w2 = jnp.transpose(W2.reshape(_OUT, _HID)).astype(jnp.bfloat16)

    out = pl.pallas_call(
        _head_kernel,
        grid=(_B // _IMGS,),
        in_specs=[
            pl.BlockSpec((_IMGS, 3, _SY + 2, _SX, _CIN),
                         lambda b: (b, 0, 0, 0, 0)),
            pl.BlockSpec((9 * _CPAD, _HID), lambda b: (0, 0)),
            pl.BlockSpec((1, _HID), lambda b: (0, 0)),
            pl.BlockSpec((1, _HID), lambda b: (0, 0)),
            pl.BlockSpec((1, _HID), lambda b: (0, 0)),
            pl.BlockSpec((1, _HID), lambda b: (0, 0)),
            pl.BlockSpec((_HID, _OUT), lambda b: (0, 0)),
            pl.BlockSpec((1, _OUT), lambda b: (0, 0)),
        ],
        out_specs=pl.BlockSpec((_IMGS, _SY, _SX, _OUT),
                               lambda b: (b, 0, 0, 0)),
        out_shape=jax.ShapeDtypeStruct((_B, _SY, _SX, _OUT), jnp.float32),
        compiler_params=pltpu.CompilerParams(
            dimension_semantics=("parallel",)),
    )(xs, w1,
      gamma.reshape(1, _HID), beta.reshape(1, _HID),
      running_mean.reshape(1, _HID), running_var.reshape(1, _HID),
      w2, b2.reshape(1, _OUT))
    return out


# speedup vs baseline: 1.0876x; 1.0876x over previous
import jax
import jax.numpy as jnp
from jax.experimental import pallas as pl
from jax.experimental.pallas import tpu as pltpu

_B, _CIN, _SY, _SX = 8, 96, 32, 32
_IMGS = 2
_CPAD = 128
_HID = 1024
_OUT = 425
_PIX = _SY * _SX
_EPS = 1e-5


def _head_kernel(x_ref, w1_ref, g_ref, b_ref, m_ref, v_ref, w2_ref, b2_ref,
                 o_ref):
    cols = []
    for i in range(_IMGS):
        pieces = []
        for dy in range(3):
            for dx in range(3):
                patch = x_ref[i, dx, dy:dy + _SY, :, :].reshape(_PIX, _CIN)
                pieces.append(patch)
        cols.append(jnp.concatenate(pieces, axis=1))
    col = jnp.concatenate(cols, axis=0)
    acc = jnp.dot(col, w1_ref[...], preferred_element_type=jnp.float32)
    scale = g_ref[...] * jax.lax.rsqrt(v_ref[...] + _EPS)
    shift = b_ref[...] - m_ref[...] * scale
    h = acc * scale + shift
    h = jnp.where(h >= 0, h, 0.1 * h)
    out = jnp.dot(h.astype(jnp.bfloat16), w2_ref[...],
                  preferred_element_type=jnp.float32) + b2_ref[...]
    o_ref[...] = out.reshape(_IMGS, _SY, _SX, _OUT)


def kernel(x, W1, gamma, beta, running_mean, running_var, W2, b2):
    xp = jnp.transpose(x, (0, 2, 3, 1))
    xp = jnp.pad(xp, ((0, 0), (1, 1), (1, 1), (0, 0))).astype(jnp.bfloat16)
    xs = jnp.stack([xp[:, :, dx:dx + _SX, :] for dx in range(3)], axis=1)

    w1 = jnp.transpose(W1, (2, 3, 1, 0)).reshape(9 * _CIN, _HID)
    w1 = w1.astype(jnp.bfloat16)
    w2 = jnp.transpose(W2.reshape(_OUT, _HID)).astype(jnp.bfloat16)

    out = pl.pallas_call(
        _head_kernel,
        grid=(_B // _IMGS,),
        in_specs=[
            pl.BlockSpec((_IMGS, 3, _SY + 2, _SX, _CIN),
                         lambda b: (b, 0, 0, 0, 0)),
            pl.BlockSpec((9 * _CIN, _HID), lambda b: (0, 0)),
            pl.BlockSpec((1, _HID), lambda b: (0, 0)),
            pl.BlockSpec((1, _HID), lambda b: (0, 0)),
            pl.BlockSpec((1, _HID), lambda b: (0, 0)),
            pl.BlockSpec((1, _HID), lambda b: (0, 0)),
            pl.BlockSpec((_HID, _OUT), lambda b: (0, 0)),
            pl.BlockSpec((1, _OUT), lambda b: (0, 0)),
        ],
        out_specs=pl.BlockSpec((_IMGS, _SY, _SX, _OUT),
                               lambda b: (b, 0, 0, 0)),
        out_shape=jax.ShapeDtypeStruct((_B, _SY, _SX, _OUT), jnp.float32),
        compiler_params=pltpu.CompilerParams(
            dimension_semantics=("parallel",)),
    )(xs, w1,
      gamma.reshape(1, _HID), beta.reshape(1, _HID),
      running_mean.reshape(1, _HID), running_var.reshape(1, _HID),
      w2, b2.reshape(1, _OUT))
    return out


# speedup vs baseline: 1.1870x; 1.0914x over previous
import jax
import jax.numpy as jnp
import numpy as np
from jax.experimental import pallas as pl
from jax.experimental.pallas import tpu as pltpu

_B, _CIN, _SY, _SX = 8, 96, 32, 32
_IMGS = 2
_HID = 1024
_OUT = 425
_PIX = _SY * _SX
_EPS = 1e-5
_MARGIN = 40
_SROWS = _MARGIN + _PIX + _MARGIN


def _head_kernel(x_ref, eye_ref, w1_ref, g_ref, b_ref, m_ref, v_ref, w2_ref,
                 b2_ref, o_ref, scr_ref):
    xpos = jax.lax.broadcasted_iota(jnp.int32, (_PIX, 1), 0) % _SX
    scr_ref[0:_MARGIN, :] = jnp.zeros((_MARGIN, _CIN), jnp.bfloat16)
    scr_ref[_MARGIN + _PIX:, :] = jnp.zeros((_MARGIN, _CIN), jnp.bfloat16)
    cols = []
    for i in range(_IMGS):
        xt = jax.lax.dot_general(x_ref[i], eye_ref[...],
                                 (((0,), (0,)), ((), ())),
                                 preferred_element_type=jnp.float32)
        scr_ref[_MARGIN:_MARGIN + _PIX, :] = xt.astype(jnp.bfloat16)
        pieces = []
        for dy in range(3):
            for dx in range(3):
                off = _MARGIN + _SX * (dy - 1) + (dx - 1)
                tap = scr_ref[off:off + _PIX, :]
                if dx == 0:
                    tap = jnp.where(xpos != 0, tap, 0)
                elif dx == 2:
                    tap = jnp.where(xpos != _SX - 1, tap, 0)
                pieces.append(tap)
        cols.append(jnp.concatenate(pieces, axis=1))
    col = jnp.concatenate(cols, axis=0)
    acc = jnp.dot(col, w1_ref[...], preferred_element_type=jnp.float32)
    scale = g_ref[...] * jax.lax.rsqrt(v_ref[...] + _EPS)
    shift = b_ref[...] - m_ref[...] * scale
    h = acc * scale + shift
    h = jnp.where(h >= 0, h, 0.1 * h)
    out = jax.lax.dot_general(h.astype(jnp.bfloat16), w2_ref[...],
                              (((1,), (1,)), ((), ())),
                              preferred_element_type=jnp.float32)
    o_ref[...] = (out + b2_ref[...]).reshape(_IMGS, _SY, _SX, _OUT)


def kernel(x, W1, gamma, beta, running_mean, running_var, W2, b2):
    xr = x.reshape(_B, _CIN, _PIX)
    w1 = jnp.transpose(W1, (2, 3, 1, 0)).reshape(9 * _CIN, _HID)
    w1 = w1.astype(jnp.bfloat16)
    w2 = W2.reshape(_OUT, _HID).astype(jnp.bfloat16)
    eye = np.eye(_CIN, dtype=np.float32)

    out = pl.pallas_call(
        _head_kernel,
        grid=(_B // _IMGS,),
        in_specs=[
            pl.BlockSpec((_IMGS, _CIN, _PIX), lambda b: (b, 0, 0)),
            pl.BlockSpec((_CIN, _CIN), lambda b: (0, 0)),
            pl.BlockSpec((9 * _CIN, _HID), lambda b: (0, 0)),
            pl.BlockSpec((1, _HID), lambda b: (0, 0)),
            pl.BlockSpec((1, _HID), lambda b: (0, 0)),
            pl.BlockSpec((1, _HID), lambda b: (0, 0)),
            pl.BlockSpec((1, _HID), lambda b: (0, 0)),
            pl.BlockSpec((_OUT, _HID), lambda b: (0, 0)),
            pl.BlockSpec((1, _OUT), lambda b: (0, 0)),
        ],
        out_specs=pl.BlockSpec((_IMGS, _SY, _SX, _OUT),
                               lambda b: (b, 0, 0, 0)),
        out_shape=jax.ShapeDtypeStruct((_B, _SY, _SX, _OUT), jnp.float32),
        scratch_shapes=[pltpu.VMEM((_SROWS, _CIN), jnp.bfloat16)],
        compiler_params=pltpu.CompilerParams(
            dimension_semantics=("parallel",)),
    )(xr, eye, w1,
      gamma.reshape(1, _HID), beta.reshape(1, _HID),
      running_mean.reshape(1, _HID), running_var.reshape(1, _HID),
      w2, b2.reshape(1, _OUT))
    return out
